# baseline (speedup 1.0000x reference)
import jax
import jax.numpy as jnp
from jax.experimental import pallas as pl
from jax.experimental.pallas import tpu as pltpu

_KNN = 4


def _gcn_kernel(x_ref, w1_ref, b1_ref, w2_ref, b2_ref, out_ref):
    n = x_ref.shape[1]
    x = x_ref[0]

    nrm = jnp.sqrt(jnp.sum(x * x, axis=1, keepdims=True))
    normf = x / jnp.maximum(nrm, 1e-12)

    sim = jax.lax.dot_general(
        normf, normf, (((1,), (1,)), ((), ())),
        preferred_element_type=jnp.float32,
    )

    rowi = jax.lax.broadcasted_iota(jnp.int32, (n, n), 0)
    coli = jax.lax.broadcasted_iota(jnp.int32, (n, n), 1)
    self_onehot = rowi == coli
    rowf = rowi.astype(jnp.float32)
    sim = jnp.where(self_onehot, -jnp.inf, sim)
    for _ in range(_KNN):
        m = jnp.max(sim, axis=0, keepdims=True)
        first = jnp.min(jnp.where(sim == m, rowf, float(n)), axis=0,
                        keepdims=True)
        sim = jnp.where(rowf == first, -jnp.inf, sim)
    a = jnp.where((sim == -jnp.inf) & ~self_onehot, 1.0, 0.0)

    ones_col = jnp.ones((n, 1), jnp.float32)
    ones_row = jnp.ones((1, n), jnp.float32)
    deg_col = jax.lax.dot_general(
        a, ones_col, (((1,), (0,)), ((), ())),
        preferred_element_type=jnp.float32,
    ) + 2.0
    deg_row = jax.lax.dot_general(
        ones_row, a, (((1,), (1,)), ((), ())),
        preferred_element_type=jnp.float32,
    ) + 2.0
    dinv = jax.lax.rsqrt(deg_col)
    dinv2 = dinv * dinv
    asc = a * jax.lax.rsqrt(deg_row)

    def layer(h_in, w_ref, b_ref):
        h = jax.lax.dot_general(
            h_in, w_ref[...], (((1,), (0,)), ((), ())),
            preferred_element_type=jnp.float32,
        )
        agg = jax.lax.dot_general(
            asc, h, (((1,), (0,)), ((), ())),
            preferred_element_type=jnp.float32,
        )
        out = dinv * agg + 2.0 * dinv2 * h + b_ref[...]
        return jnp.maximum(out, 0.0)

    y1 = layer(x, w1_ref, b1_ref)
    y2 = layer(y1, w2_ref, b2_ref)
    out_ref[0] = y2


def kernel(feature_maps, W1, b1, W2, b2):
    B, C, H, Wd = feature_maps.shape
    n = H * Wd
    hid = W1.shape[1]
    out_c = W2.shape[1]
    x = jnp.transpose(feature_maps, (0, 2, 3, 1)).reshape(B, n, C)
    b1r = b1.reshape(1, hid)
    b2r = b2.reshape(1, out_c)

    y = pl.pallas_call(
        _gcn_kernel,
        grid=(B,),
        in_specs=[
            pl.BlockSpec((1, n, C), lambda b: (b, 0, 0)),
            pl.BlockSpec((C, hid), lambda b: (0, 0)),
            pl.BlockSpec((1, hid), lambda b: (0, 0)),
            pl.BlockSpec((hid, out_c), lambda b: (0, 0)),
            pl.BlockSpec((1, out_c), lambda b: (0, 0)),
        ],
        out_specs=pl.BlockSpec((1, n, out_c), lambda b: (b, 0, 0)),
        out_shape=jax.ShapeDtypeStruct((B, n, out_c), jnp.float32),
        compiler_params=pltpu.CompilerParams(
            dimension_semantics=("parallel",)),
    )(x, W1, b1r, W2, b2r)

    return jnp.transpose(y.reshape(B, H, Wd, out_c), (0, 3, 1, 2))


# speedup vs baseline: 1.0517x; 1.0517x over previous
import jax
import jax.numpy as jnp
from jax.experimental import pallas as pl
from jax.experimental.pallas import tpu as pltpu

_KNN = 4


def _gcn_kernel(x_ref, w1_ref, b1_ref, w2_ref, b2_ref, out_ref):
    n = x_ref.shape[1]
    x = x_ref[0]

    nrm = jnp.sqrt(jnp.sum(x * x, axis=1, keepdims=True))
    normf = x / jnp.maximum(nrm, 1e-12)

    sim = jax.lax.dot_general(
        normf, normf, (((1,), (1,)), ((), ())),
        preferred_element_type=jnp.float32,
    )

    rowi = jax.lax.broadcasted_iota(jnp.int32, (n, n), 0)
    coli = jax.lax.broadcasted_iota(jnp.int32, (n, n), 1)
    self_onehot = rowi == coli
    colf = coli.astype(jnp.float32)
    rowf_col = jax.lax.broadcasted_iota(jnp.int32, (n, 1), 0).astype(
        jnp.float32)
    first = rowf_col
    for _ in range(_KNN):
        sim = jnp.where(colf == first, -jnp.inf, sim)
        m = jnp.max(sim, axis=1, keepdims=True)
        first = jnp.min(jnp.where(sim == m, colf, float(n)), axis=1,
                        keepdims=True)
    a = jnp.where(((sim == -jnp.inf) & ~self_onehot) | (colf == first),
                  1.0, 0.0)

    ones_col = jnp.ones((n, 1), jnp.float32)
    deg = jax.lax.dot_general(
        a, ones_col, (((0,), (0,)), ((), ())),
        preferred_element_type=jnp.float32,
    ) + 2.0
    dinv = jax.lax.rsqrt(deg)
    dinv2 = dinv * dinv

    def layer(h_in, w_ref, b_ref):
        h = jax.lax.dot_general(
            h_in, w_ref[...], (((1,), (0,)), ((), ())),
            preferred_element_type=jnp.float32,
        )
        g = dinv * h
        agg = jax.lax.dot_general(
            a, g, (((0,), (0,)), ((), ())),
            preferred_element_type=jnp.float32,
        )
        out = dinv * agg + 2.0 * dinv2 * h + b_ref[...]
        return jnp.maximum(out, 0.0)

    y1 = layer(x, w1_ref, b1_ref)
    y2 = layer(y1, w2_ref, b2_ref)
    out_ref[0] = y2


def kernel(feature_maps, W1, b1, W2, b2):
    B, C, H, Wd = feature_maps.shape
    n = H * Wd
    hid = W1.shape[1]
    out_c = W2.shape[1]
    x = jnp.transpose(feature_maps, (0, 2, 3, 1)).reshape(B, n, C)
    b1r = b1.reshape(1, hid)
    b2r = b2.reshape(1, out_c)

    y = pl.pallas_call(
        _gcn_kernel,
        grid=(B,),
        in_specs=[
            pl.BlockSpec((1, n, C), lambda b: (b, 0, 0)),
            pl.BlockSpec((C, hid), lambda b: (0, 0)),
            pl.BlockSpec((1, hid), lambda b: (0, 0)),
            pl.BlockSpec((hid, out_c), lambda b: (0, 0)),
            pl.BlockSpec((1, out_c), lambda b: (0, 0)),
        ],
        out_specs=pl.BlockSpec((1, n, out_c), lambda b: (b, 0, 0)),
        out_shape=jax.ShapeDtypeStruct((B, n, out_c), jnp.float32),
        compiler_params=pltpu.CompilerParams(
            dimension_semantics=("parallel",)),
    )(x, W1, b1r, W2, b2r)

    return jnp.transpose(y.reshape(B, H, Wd, out_c), (0, 3, 1, 2))


# speedup vs baseline: 1.0860x; 1.0326x over previous
import jax
import jax.numpy as jnp
from jax.experimental import pallas as pl
from jax.experimental.pallas import tpu as pltpu

_KNN = 4


def _gcn_kernel(x_ref, w1_ref, b1_ref, w2_ref, b2_ref, out_ref):
    n = x_ref.shape[1]
    x = x_ref[0]

    nrm = jnp.sqrt(jnp.sum(x * x, axis=1, keepdims=True))
    normf = x / jnp.maximum(nrm, 1e-12)

    sim = jax.lax.dot_general(
        normf, normf, (((1,), (1,)), ((), ())),
        preferred_element_type=jnp.float32,
    )

    rowi = jax.lax.broadcasted_iota(jnp.int32, (n, n), 0)
    coli = jax.lax.broadcasted_iota(jnp.int32, (n, n), 1)
    self_onehot = rowi == coli
    colf = coli.astype(jnp.float32)
    sim = jnp.where(self_onehot, -jnp.inf, sim)
    for _ in range(_KNN):
        m = jnp.max(sim, axis=1, keepdims=True)
        first = jnp.min(jnp.where(sim == m, colf, float(n)), axis=1,
                        keepdims=True)
        sim = jnp.where(colf == first, -jnp.inf, sim)
    a = jnp.where(self_onehot, 2.0, jnp.where(sim == -jnp.inf, 1.0, 0.0))

    ones_col = jnp.ones((n, 1), jnp.float32)
    deg = jax.lax.dot_general(
        a, ones_col, (((0,), (0,)), ((), ())),
        preferred_element_type=jnp.float32,
    )
    dinv = jax.lax.rsqrt(deg)

    def layer(h_in, w_ref, b_ref):
        h = jax.lax.dot_general(
            h_in, w_ref[...], (((1,), (0,)), ((), ())),
            preferred_element_type=jnp.float32,
        )
        g = dinv * h
        agg = jax.lax.dot_general(
            a, g, (((0,), (0,)), ((), ())),
            preferred_element_type=jnp.float32,
        )
        out = dinv * agg + b_ref[...]
        return jnp.maximum(out, 0.0)

    y1 = layer(x, w1_ref, b1_ref)
    y2 = layer(y1, w2_ref, b2_ref)
    out_ref[0] = y2


def kernel(feature_maps, W1, b1, W2, b2):
    B, C, H, Wd = feature_maps.shape
    n = H * Wd
    hid = W1.shape[1]
    out_c = W2.shape[1]
    x = jnp.transpose(feature_maps, (0, 2, 3, 1)).reshape(B, n, C)
    b1r = b1.reshape(1, hid)
    b2r = b2.reshape(1, out_c)

    y = pl.pallas_call(
        _gcn_kernel,
        grid=(B,),
        in_specs=[
            pl.BlockSpec((1, n, C), lambda b: (b, 0, 0)),
            pl.BlockSpec((C, hid), lambda b: (0, 0)),
            pl.BlockSpec((1, hid), lambda b: (0, 0)),
            pl.BlockSpec((hid, out_c), lambda b: (0, 0)),
            pl.BlockSpec((1, out_c), lambda b: (0, 0)),
        ],
        out_specs=pl.BlockSpec((1, n, out_c), lambda b: (b, 0, 0)),
        out_shape=jax.ShapeDtypeStruct((B, n, out_c), jnp.float32),
        compiler_params=pltpu.CompilerParams(
            dimension_semantics=("parallel",)),
    )(x, W1, b1r, W2, b2r)

    return jnp.transpose(y.reshape(B, H, Wd, out_c), (0, 3, 1, 2))


# speedup vs baseline: 1.1610x; 1.0690x over previous
import jax
import jax.numpy as jnp
from jax.experimental import pallas as pl
from jax.experimental.pallas import tpu as pltpu

_KNN = 4


def _gcn_kernel(x_ref, w1_ref, b1_ref, w2_ref, b2_ref, out_ref):
    for img in range(x_ref.shape[0]):
        _gcn_one(x_ref, w1_ref, b1_ref, w2_ref, b2_ref, out_ref, img)


def _gcn_one(x_ref, w1_ref, b1_ref, w2_ref, b2_ref, out_ref, img):
    n = x_ref.shape[1]
    x = x_ref[img]

    nrm = jnp.sqrt(jnp.sum(x * x, axis=1, keepdims=True))
    normf = x / jnp.maximum(nrm, 1e-12)

    sim = jax.lax.dot_general(
        normf, normf, (((1,), (1,)), ((), ())),
        preferred_element_type=jnp.float32,
    )

    rowi = jax.lax.broadcasted_iota(jnp.int32, (n, n), 0)
    coli = jax.lax.broadcasted_iota(jnp.int32, (n, n), 1)
    self_onehot = rowi == coli
    colf = coli.astype(jnp.float32)
    sim = jnp.where(self_onehot, -jnp.inf, sim)
    for _ in range(_KNN):
        m = jnp.max(sim, axis=1, keepdims=True)
        first = jnp.min(jnp.where(sim == m, colf, float(n)), axis=1,
                        keepdims=True)
        sim = jnp.where(colf == first, -jnp.inf, sim)
    a = jnp.where((sim == -jnp.inf) & ~self_onehot, 1.0, 0.0)

    ones_col = jnp.ones((n, 1), jnp.float32)
    deg = jax.lax.dot_general(
        a, ones_col, (((0,), (0,)), ((), ())),
        preferred_element_type=jnp.float32,
    ) + 2.0
    dinv = jax.lax.rsqrt(deg)
    dinv2 = dinv * dinv

    def layer(h_in, w_ref, b_ref):
        h = jax.lax.dot_general(
            h_in, w_ref[...], (((1,), (0,)), ((), ())),
            preferred_element_type=jnp.float32,
        )
        g = dinv * h
        agg = jax.lax.dot_general(
            a, g, (((0,), (0,)), ((), ())),
            preferred_element_type=jnp.float32,
        )
        out = dinv * agg + 2.0 * dinv2 * h + b_ref[...]
        return jnp.maximum(out, 0.0)

    y1 = layer(x, w1_ref, b1_ref)
    y2 = layer(y1, w2_ref, b2_ref)
    out_ref[img] = y2


def kernel(feature_maps, W1, b1, W2, b2):
    B, C, H, Wd = feature_maps.shape
    n = H * Wd
    hid = W1.shape[1]
    out_c = W2.shape[1]
    x = jnp.transpose(feature_maps, (0, 2, 3, 1)).reshape(B, n, C)
    b1r = b1.reshape(1, hid)
    b2r = b2.reshape(1, out_c)

    y = pl.pallas_call(
        _gcn_kernel,
        grid=(B // 2,),
        in_specs=[
            pl.BlockSpec((2, n, C), lambda b: (b, 0, 0)),
            pl.BlockSpec((C, hid), lambda b: (0, 0)),
            pl.BlockSpec((1, hid), lambda b: (0, 0)),
            pl.BlockSpec((hid, out_c), lambda b: (0, 0)),
            pl.BlockSpec((1, out_c), lambda b: (0, 0)),
        ],
        out_specs=pl.BlockSpec((2, n, out_c), lambda b: (b, 0, 0)),
        out_shape=jax.ShapeDtypeStruct((B, n, out_c), jnp.float32),
        compiler_params=pltpu.CompilerParams(
            dimension_semantics=("parallel",)),
    )(x, W1, b1r, W2, b2r)

    return jnp.transpose(y.reshape(B, H, Wd, out_c), (0, 3, 1, 2))


# speedup vs baseline: 1.1860x; 1.0215x over previous
import jax
import jax.numpy as jnp
from jax.experimental import pallas as pl
from jax.experimental.pallas import tpu as pltpu

_KNN = 4


def _gcn_kernel(x_ref, w1_ref, b1_ref, w2_ref, b2_ref, out_ref):
    for img in range(x_ref.shape[0]):
        _gcn_one(x_ref, w1_ref, b1_ref, w2_ref, b2_ref, out_ref, img)


def _gcn_one(x_ref, w1_ref, b1_ref, w2_ref, b2_ref, out_ref, img):
    n = x_ref.shape[1]
    x = x_ref[img]

    nrm = jnp.sqrt(jnp.sum(x * x, axis=1, keepdims=True))
    normf = x / jnp.maximum(nrm, 1e-12)

    sim = jax.lax.dot_general(
        normf, normf, (((1,), (1,)), ((), ())),
        preferred_element_type=jnp.float32,
    )

    rowi = jax.lax.broadcasted_iota(jnp.int32, (n, n), 0)
    coli = jax.lax.broadcasted_iota(jnp.int32, (n, n), 1)
    self_onehot = rowi == coli
    colf = coli.astype(jnp.float32)
    sim = jnp.where(self_onehot, -jnp.inf, sim)
    for _ in range(_KNN):
        m = jnp.max(sim, axis=1, keepdims=True)
        first = jnp.min(jnp.where(sim == m, colf, float(n)), axis=1,
                        keepdims=True)
        sim = jnp.where(colf == first, -jnp.inf, sim)
    a = jnp.where((sim == -jnp.inf) & ~self_onehot, 1.0, 0.0)

    ones_col = jnp.ones((n, 1), jnp.float32)
    deg = jax.lax.dot_general(
        a, ones_col, (((0,), (0,)), ((), ())),
        preferred_element_type=jnp.float32,
    ) + 2.0
    dinv = jax.lax.rsqrt(deg)
    dinv2 = dinv * dinv

    def layer(h_in, w_ref, b_ref):
        h = jax.lax.dot_general(
            h_in, w_ref[...], (((1,), (0,)), ((), ())),
            preferred_element_type=jnp.float32,
        )
        g = dinv * h
        agg = jax.lax.dot_general(
            a, g, (((0,), (0,)), ((), ())),
            preferred_element_type=jnp.float32,
        )
        out = dinv * agg + 2.0 * dinv2 * h + b_ref[...]
        return jnp.maximum(out, 0.0)

    y1 = layer(x, w1_ref, b1_ref)
    y2 = layer(y1, w2_ref, b2_ref)
    out_ref[img] = y2


def kernel(feature_maps, W1, b1, W2, b2):
    B, C, H, Wd = feature_maps.shape
    n = H * Wd
    hid = W1.shape[1]
    out_c = W2.shape[1]
    x = jnp.transpose(feature_maps, (0, 2, 3, 1)).reshape(B, n, C)
    b1r = b1.reshape(1, hid)
    b2r = b2.reshape(1, out_c)

    y = pl.pallas_call(
        _gcn_kernel,
        grid=(B // 4,),
        in_specs=[
            pl.BlockSpec((4, n, C), lambda b: (b, 0, 0)),
            pl.BlockSpec((C, hid), lambda b: (0, 0)),
            pl.BlockSpec((1, hid), lambda b: (0, 0)),
            pl.BlockSpec((hid, out_c), lambda b: (0, 0)),
            pl.BlockSpec((1, out_c), lambda b: (0, 0)),
        ],
        out_specs=pl.BlockSpec((4, n, out_c), lambda b: (b, 0, 0)),
        out_shape=jax.ShapeDtypeStruct((B, n, out_c), jnp.float32),
        compiler_params=pltpu.CompilerParams(
            dimension_semantics=("parallel",)),
    )(x, W1, b1r, W2, b2r)

    return jnp.transpose(y.reshape(B, H, Wd, out_c), (0, 3, 1, 2))
